# baseline (speedup 1.0000x reference)
import functools

import jax
import jax.numpy as jnp
from jax import lax
from jax.experimental import pallas as pl
from jax.experimental.pallas import tpu as pltpu
from jax.experimental.pallas import tpu_sc as plsc

N = 10000
N_PAD = 10240
E = 320000
NC = 2
NS = 16
NW = NC * NS
G = 128
GPW = 79
GPA = 80
CH = 8
EPW = GPW * G
EPW_A = GPA * G
EP = NW * EPW
EPA = NW * EPW_A
RPS = N_PAD // NS
CHUNKS = (128, 128, 128, 128, 128)

_mesh = lambda: plsc.VectorSubcoreMesh(core_axis_name="c", subcore_axis_name="s")

_GDN = lax.GatherDimensionNumbers(
    offset_dims=(), collapsed_slice_dims=(0,), start_index_map=(0,))


def _splat(vec16, l):
    return lax.gather(vec16, jnp.full((16, 1), l, jnp.int32), _GDN, (1,),
                      mode=lax.GatherScatterMode.PROMISE_IN_BOUNDS)


@functools.partial(
    pl.kernel,
    out_type=jax.ShapeDtypeStruct((NC, N_PAD // 16, 16), jnp.float32),
    mesh=_mesh(),
    scratch_types=[
        pltpu.VMEM((EPW,), jnp.int32),
        pltpu.VMEM((EPW,), jnp.float32),
        pltpu.VMEM((GPW, G), jnp.int32),
        pltpu.VMEM((G, 16), jnp.float32),
        pltpu.VMEM_SHARED((N_PAD // 16, 16), jnp.float32),
    ],
)
def _deg(colf_hbm, ewf_hbm, out_hbm, col_v, ew_v, cidx_v, rows_v, acc_sh):
    cid = lax.axis_index("c")
    sid = lax.axis_index("s")
    wid = sid * NC + cid
    lanes = lax.iota(jnp.int32, 16)

    pltpu.sync_copy(colf_hbm.at[pl.ds(wid * EPW_A, EPW)], col_v)
    pltpu.sync_copy(ewf_hbm.at[pl.ds(wid * EPW_A, EPW)], ew_v)

    def zero_body(i, _):
        rows_v[i, :] = jnp.zeros((16,), jnp.float32)
        return 0

    lax.fori_loop(0, G, zero_body, 0)
    nrps = N_PAD // 16 // NS
    pltpu.sync_copy(rows_v.at[pl.ds(0, nrps)],
                    acc_sh.at[pl.ds(sid * nrps, nrps)])

    def cidx_body(j, _):
        for sg in range(G // 16):
            sl = pl.ds(sg * 16, 16)
            cidx_v[j, sl] = col_v[pl.ds(j * G + sg * 16, 16)] >> 4
        return 0

    lax.fori_loop(0, GPW, cidx_body, 0)
    plsc.subcore_barrier()

    def outer(j, _):
        def build(sg, _2):
            e0 = j * G + sg * 16
            cv = col_v[pl.ds(e0, 16)]
            wv = ew_v[pl.ds(e0, 16)]
            lo = cv & 15
            for l in range(16):
                oh = jnp.where(lanes == _splat(lo, l),
                               jnp.float32(1.0), jnp.float32(0.0))
                rows_v[sg * 16 + l, :] = _splat(wv, l) * oh
            return 0

        lax.fori_loop(0, G // 16, build, 0)
        pltpu.sync_copy(rows_v, acc_sh.at[cidx_v.at[j]], add=True)
        return 0

    lax.fori_loop(0, GPW, outer, 0)
    plsc.subcore_barrier()

    pltpu.sync_copy(acc_sh.at[pl.ds(sid * nrps, nrps)],
                    rows_v.at[pl.ds(0, nrps)])
    pltpu.sync_copy(rows_v.at[pl.ds(0, nrps)],
                    out_hbm.at[cid, pl.ds(sid * nrps, nrps)])


def _make_spmm(D):
    NB = 2 if D == 128 else 4

    @functools.partial(
        pl.kernel,
        out_type=jax.ShapeDtypeStruct((NC, N_PAD, D), jnp.float32),
        mesh=_mesh(),
        compiler_params=pltpu.CompilerParams(use_tc_tiling_on_sc=(D == 128)),
        scratch_types=[
            pltpu.VMEM((2 * CH * G,), jnp.int32),
            pltpu.VMEM((2 * CH, G), jnp.int32),
            pltpu.VMEM((2 * CH * G,), jnp.float32),
        ] + [pltpu.VMEM((G, D), jnp.float32) for _ in range(NB)]
        + [pltpu.VMEM_SHARED((N_PAD, D), jnp.float32)]
        + [pltpu.SemaphoreType.DMA for _ in range(NB)],
    )
    def spmm(rowf_hbm, col3_hbm, ewf_hbm, h_hbm, out_hbm,
             row_v, col_v, ew_v, *rest):
        bufs = rest[:NB]
        acc_sh = rest[NB]
        sems = rest[NB + 1:]
        cid = lax.axis_index("c")
        sid = lax.axis_index("s")
        wid = sid * NC + cid

        def zrows(i, _):
            for cch in range(D // 16):
                bufs[0][i, pl.ds(cch * 16, 16)] = jnp.zeros((16,), jnp.float32)
            return 0

        lax.fori_loop(0, G, zrows, 0)

        base = sid * RPS
        off = 0
        for sz in CHUNKS:
            pltpu.sync_copy(bufs[0].at[pl.ds(0, sz)],
                            acc_sh.at[pl.ds(base + off, sz)])
            off += sz

        plsc.subcore_barrier()

        def refill(jj):
            c = jj // CH
            par = c % 2
            src0 = wid * EPW_A + c * CH * G
            pltpu.sync_copy(rowf_hbm.at[pl.ds(src0, CH * G)],
                            row_v.at[pl.ds(par * CH * G, CH * G)])
            pltpu.sync_copy(col3_hbm.at[wid, pl.ds(c * CH, CH)],
                            col_v.at[pl.ds(par * CH, CH)])
            pltpu.sync_copy(ewf_hbm.at[pl.ds(src0, CH * G)],
                            ew_v.at[pl.ds(par * CH * G, CH * G)])

        def eoff(j):
            return ((j // CH) % 2 * CH + j % CH) * G

        def gstart(jj, buf, sem):
            @pl.when(jj % CH == 0)
            def _():
                refill(jj)

            pltpu.async_copy(h_hbm.at[row_v.at[pl.ds(eoff(jj), G)]],
                             buf, sem)

        def gwait(buf, sem):
            pltpu.make_async_copy(h_hbm.at[pl.ds(0, G)], buf, sem).wait()

        def scale(buf, j):
            def sbody(sg, _):
                wv = ew_v[pl.ds(eoff(j) + sg * 16, 16)]
                for l in range(16):
                    spl = _splat(wv, l)
                    for cch in range(D // 16):
                        sl2 = pl.ds(cch * 16, 16)
                        buf[sg * 16 + l, sl2] = buf[sg * 16 + l, sl2] * spl
                return 0

            lax.fori_loop(0, G // 16, sbody, 0)

        def scat(buf, j):
            pltpu.sync_copy(buf,
                            acc_sh.at[col_v.at[(j // CH) % 2 * CH + j % CH]],
                            add=True)

        for i in range(NB):
            gstart(i, bufs[i], sems[i])

        def body(k, _):
            for i in range(NB):
                j = k * NB + i
                gwait(bufs[i], sems[i])
                scale(bufs[i], j)
                scat(bufs[i], j)
                jn = j + NB

                @pl.when(jn < GPW)
                def _():
                    gstart(jn, bufs[i], sems[i])

            return 0

        lax.fori_loop(0, GPW // NB, body, 0)
        for i in range(GPW - (GPW // NB) * NB):
            j = (GPW // NB) * NB + i
            gwait(bufs[i], sems[i])
            scale(bufs[i], j)
            scat(bufs[i], j)
        plsc.subcore_barrier()

        off = 0
        for sz in CHUNKS:
            pltpu.sync_copy(acc_sh.at[pl.ds(base + off, sz)],
                            out_hbm.at[cid, pl.ds(base + off, sz)])
            off += sz

    return spmm


_spmm128 = _make_spmm(128)


_RB = 1000


def _matmul_scale(x, W, deg_col):
    din, dout = W.shape

    def body(x_ref, w_ref, d_ref, o_ref):
        dis = lax.rsqrt(d_ref[0] + d_ref[1] + 1.0)
        o_ref[...] = dis * jnp.dot(x_ref[...], w_ref[...],
                                   preferred_element_type=jnp.float32)

    return pl.pallas_call(
        body,
        grid=(N // _RB,),
        in_specs=[
            pl.BlockSpec((_RB, din), lambda i: (i, 0)),
            pl.BlockSpec((din, dout), lambda i: (0, 0)),
            pl.BlockSpec((2, _RB, 1), lambda i: (0, i, 0)),
        ],
        out_specs=pl.BlockSpec((_RB, dout), lambda i: (i, 0)),
        out_shape=jax.ShapeDtypeStruct((N, dout), jnp.float32),
    )(x, W, deg_col)


def _layer1_out(acc, hp, deg_col, b1, W2):
    def body(a_ref, h_ref, d_ref, b_ref, w_ref, o_ref):
        dis = lax.rsqrt(d_ref[0] + d_ref[1] + 1.0)
        z = dis * (a_ref[0] + a_ref[1] + h_ref[...]) + b_ref[...]
        h = jnp.maximum(z, 0.0)
        hw = dis * jnp.dot(h, w_ref[...], preferred_element_type=jnp.float32)
        o_ref[...] = jnp.concatenate([hw, jnp.zeros_like(hw)], axis=1)

    return pl.pallas_call(
        body,
        grid=(N // _RB,),
        in_specs=[
            pl.BlockSpec((2, _RB, 128), lambda i: (0, i, 0)),
            pl.BlockSpec((_RB, 128), lambda i: (i, 0)),
            pl.BlockSpec((2, _RB, 1), lambda i: (0, i, 0)),
            pl.BlockSpec((128,), lambda i: (0,)),
            pl.BlockSpec((128, 64), lambda i: (0, 0)),
        ],
        out_specs=pl.BlockSpec((_RB, 128), lambda i: (i, 0)),
        out_shape=jax.ShapeDtypeStruct((N, 128), jnp.float32),
    )(acc, hp, deg_col, b1, W2)


def _final_out(acc, hp, deg_col, b2):
    def body(a_ref, h_ref, d_ref, b_ref, o_ref):
        dis = lax.rsqrt(d_ref[0] + d_ref[1] + 1.0)
        z = dis * (a_ref[0, :, :64] + a_ref[1, :, :64]
                   + h_ref[:, :64]) + b_ref[...]
        m = jnp.max(z, axis=1, keepdims=True)
        s = z - m
        o_ref[...] = s - jnp.log(jnp.sum(jnp.exp(s), axis=1, keepdims=True))

    return pl.pallas_call(
        body,
        grid=(N // _RB,),
        in_specs=[
            pl.BlockSpec((2, _RB, 128), lambda i: (0, i, 0)),
            pl.BlockSpec((_RB, 128), lambda i: (i, 0)),
            pl.BlockSpec((2, _RB, 1), lambda i: (0, i, 0)),
            pl.BlockSpec((64,), lambda i: (0,)),
        ],
        out_specs=pl.BlockSpec((_RB, 64), lambda i: (i, 0)),
        out_shape=jax.ShapeDtypeStruct((N, 64), jnp.float32),
    )(acc, hp, deg_col, b2)


def kernel(x, edge_index, edge_weight, W1, b1, W2, b2):
    row = edge_index[0].astype(jnp.int32)
    col = edge_index[1].astype(jnp.int32)
    pad = EP - E

    zi = jnp.zeros((pad,), jnp.int32)
    zf = jnp.zeros((pad,), jnp.float32)
    padw = ((0, 0), (0, GPA - GPW), (0, 0))
    row3 = jnp.pad(jnp.concatenate([row, zi]).reshape(NW, GPW, G), padw)
    col3 = jnp.pad(jnp.concatenate([col, zi]).reshape(NW, GPW, G), padw)
    ew3 = jnp.pad(jnp.concatenate([edge_weight, zf]).reshape(NW, GPW, G), padw)
    rowf = row3.reshape(EPA)
    colf = col3.reshape(EPA)
    ewf = ew3.reshape(EPA)

    deg_col = _deg(colf, ewf).reshape(NC, N_PAD, 1)
    h1 = _matmul_scale(x, W1, deg_col)
    acc1 = _spmm128(rowf, col3, ewf, h1)
    h2 = _layer1_out(acc1, h1, deg_col, b1, W2)
    acc2 = _spmm128(rowf, col3, ewf, h2)
    return _final_out(acc2, h2, deg_col, b2)


# speedup vs baseline: 1.2504x; 1.2504x over previous
import functools

import jax
import jax.numpy as jnp
from jax import lax
from jax.experimental import pallas as pl
from jax.experimental.pallas import tpu as pltpu
from jax.experimental.pallas import tpu_sc as plsc

N = 10000
N_PAD = 10240
E = 320000
NC = 2
NS = 16
NW = NC * NS
G = 128
GPW = 81
GPA = 88
CH = 8
EPW = GPW * G
EPW_A = GPA * G
EP = NW * EPW
EPA = NW * EPW_A
RPS = N_PAD // NS
CHUNKS = (128, 128, 128, 128, 128)

_mesh = lambda: plsc.VectorSubcoreMesh(core_axis_name="c", subcore_axis_name="s")

_GDN = lax.GatherDimensionNumbers(
    offset_dims=(), collapsed_slice_dims=(0,), start_index_map=(0,))


def _splat(vec16, l):
    return lax.gather(vec16, jnp.full((16, 1), l, jnp.int32), _GDN, (1,),
                      mode=lax.GatherScatterMode.PROMISE_IN_BOUNDS)


@functools.partial(
    pl.kernel,
    out_type=jax.ShapeDtypeStruct((NC, N_PAD // 16, 16), jnp.float32),
    mesh=_mesh(),
    scratch_types=[
        pltpu.VMEM((EPW,), jnp.int32),
        pltpu.VMEM((EPW,), jnp.float32),
        pltpu.VMEM((GPW, G), jnp.int32),
        pltpu.VMEM((G, 16), jnp.float32),
        pltpu.VMEM_SHARED((N_PAD // 16, 16), jnp.float32),
    ],
)
def _deg(colf_hbm, ewf_hbm, out_hbm, col_v, ew_v, cidx_v, rows_v, acc_sh):
    cid = lax.axis_index("c")
    sid = lax.axis_index("s")
    wid = sid * NC + cid
    lanes = lax.iota(jnp.int32, 16)

    pltpu.sync_copy(colf_hbm.at[pl.ds(wid * EPW_A, EPW)], col_v)
    pltpu.sync_copy(ewf_hbm.at[pl.ds(wid * EPW_A, EPW)], ew_v)

    def zero_body(i, _):
        rows_v[i, :] = jnp.zeros((16,), jnp.float32)
        return 0

    lax.fori_loop(0, G, zero_body, 0)
    nrps = N_PAD // 16 // NS
    pltpu.sync_copy(rows_v.at[pl.ds(0, nrps)],
                    acc_sh.at[pl.ds(sid * nrps, nrps)])

    def cidx_body(j, _):
        for sg in range(G // 16):
            sl = pl.ds(sg * 16, 16)
            cidx_v[j, sl] = col_v[pl.ds(j * G + sg * 16, 16)] >> 4
        return 0

    lax.fori_loop(0, GPW, cidx_body, 0)
    plsc.subcore_barrier()

    def outer(j, _):
        def build(sg, _2):
            e0 = j * G + sg * 16
            cv = col_v[pl.ds(e0, 16)]
            wv = ew_v[pl.ds(e0, 16)]
            lo = cv & 15
            for l in range(16):
                oh = jnp.where(lanes == _splat(lo, l),
                               jnp.float32(1.0), jnp.float32(0.0))
                rows_v[sg * 16 + l, :] = _splat(wv, l) * oh
            return 0

        lax.fori_loop(0, G // 16, build, 0)
        pltpu.sync_copy(rows_v, acc_sh.at[cidx_v.at[j]], add=True)
        return 0

    lax.fori_loop(0, GPW, outer, 0)
    plsc.subcore_barrier()

    pltpu.sync_copy(acc_sh.at[pl.ds(sid * nrps, nrps)],
                    rows_v.at[pl.ds(0, nrps)])
    pltpu.sync_copy(rows_v.at[pl.ds(0, nrps)],
                    out_hbm.at[cid, pl.ds(sid * nrps, nrps)])


def _make_spmm(D):
    NB = 2 if D == 128 else 4

    @functools.partial(
        pl.kernel,
        out_type=jax.ShapeDtypeStruct((NC, N_PAD, D), jnp.float32),
        mesh=_mesh(),
        compiler_params=pltpu.CompilerParams(use_tc_tiling_on_sc=(D == 128)),
        scratch_types=[
            pltpu.VMEM((2 * CH * G,), jnp.int32),
            pltpu.VMEM((2 * CH, G), jnp.int32),
            pltpu.VMEM((2 * CH * G,), jnp.float32),
        ] + [pltpu.VMEM((G, D), jnp.float32) for _ in range(NB)]
        + [pltpu.VMEM_SHARED((N_PAD, D), jnp.float32)]
        + [pltpu.SemaphoreType.DMA for _ in range(NB)],
    )
    def spmm(rowf_hbm, col3_hbm, ewf_hbm, h_hbm, out_hbm,
             row_v, col_v, ew_v, *rest):
        bufs = rest[:NB]
        acc_sh = rest[NB]
        sems = rest[NB + 1:]
        cid = lax.axis_index("c")
        sid = lax.axis_index("s")
        wid = sid * NC + cid

        def zrows(i, _):
            for cch in range(D // 16):
                bufs[0][i, pl.ds(cch * 16, 16)] = jnp.zeros((16,), jnp.float32)
            return 0

        lax.fori_loop(0, G, zrows, 0)

        base = sid * RPS
        off = 0
        for sz in CHUNKS:
            pltpu.sync_copy(bufs[0].at[pl.ds(0, sz)],
                            acc_sh.at[pl.ds(base + off, sz)])
            off += sz

        plsc.subcore_barrier()

        def refill(jj):
            c = jj // CH
            par = c % 2
            src0 = wid * EPW_A + c * CH * G
            pltpu.sync_copy(rowf_hbm.at[pl.ds(src0, CH * G)],
                            row_v.at[pl.ds(par * CH * G, CH * G)])
            pltpu.sync_copy(col3_hbm.at[wid, pl.ds(c * CH, CH)],
                            col_v.at[pl.ds(par * CH, CH)])
            pltpu.sync_copy(ewf_hbm.at[pl.ds(src0, CH * G)],
                            ew_v.at[pl.ds(par * CH * G, CH * G)])

        def eoff(j):
            return ((j // CH) % 2 * CH + j % CH) * G

        def gstart(jj, buf, sem):
            @pl.when(jj % CH == 0)
            def _():
                refill(jj)

            pltpu.async_copy(h_hbm.at[row_v.at[pl.ds(eoff(jj), G)]],
                             buf, sem)

        def gwait(buf, sem):
            pltpu.make_async_copy(h_hbm.at[pl.ds(0, G)], buf, sem).wait()

        def scale(buf, j):
            def sbody(sg, _):
                wv = ew_v[pl.ds(eoff(j) + sg * 16, 16)]
                for l in range(16):
                    spl = _splat(wv, l)
                    for cch in range(D // 16):
                        sl2 = pl.ds(cch * 16, 16)
                        buf[sg * 16 + l, sl2] = buf[sg * 16 + l, sl2] * spl
                return 0

            lax.fori_loop(0, G // 16, sbody, 0)

        def scat(buf, j):
            pltpu.sync_copy(buf,
                            acc_sh.at[col_v.at[(j // CH) % 2 * CH + j % CH]],
                            add=True)

        for i in range(NB):
            gstart(i, bufs[i], sems[i])

        def body(k, _):
            for i in range(NB):
                j = k * NB + i
                gwait(bufs[i], sems[i])
                scale(bufs[i], j)
                scat(bufs[i], j)
                jn = j + NB

                @pl.when(jn < GPW)
                def _():
                    gstart(jn, bufs[i], sems[i])

            return 0

        lax.fori_loop(0, GPW // NB, body, 0)
        for i in range(GPW - (GPW // NB) * NB):
            j = (GPW // NB) * NB + i
            gwait(bufs[i], sems[i])
            scale(bufs[i], j)
            scat(bufs[i], j)
        plsc.subcore_barrier()

        off = 0
        for sz in CHUNKS:
            pltpu.sync_copy(acc_sh.at[pl.ds(base + off, sz)],
                            out_hbm.at[cid, pl.ds(base + off, sz)])
            off += sz

    return spmm


_spmm128 = _make_spmm(128)


_RB = 1000


def _matmul_scale(x, W, deg_col):
    din, dout = W.shape

    def body(x_ref, w_ref, d_ref, o_ref):
        dis = lax.rsqrt(d_ref[0] + d_ref[1])
        o_ref[...] = dis * jnp.dot(x_ref[...], w_ref[...],
                                   preferred_element_type=jnp.float32)

    return pl.pallas_call(
        body,
        grid=(N // _RB,),
        in_specs=[
            pl.BlockSpec((_RB, din), lambda i: (i, 0)),
            pl.BlockSpec((din, dout), lambda i: (0, 0)),
            pl.BlockSpec((2, _RB, 1), lambda i: (0, i, 0)),
        ],
        out_specs=pl.BlockSpec((_RB, dout), lambda i: (i, 0)),
        out_shape=jax.ShapeDtypeStruct((N, dout), jnp.float32),
    )(x, W, deg_col)


def _layer1_out(acc, deg_col, b1, W2):
    def body(a_ref, d_ref, b_ref, w_ref, o_ref):
        dis = lax.rsqrt(d_ref[0] + d_ref[1])
        z = dis * (a_ref[0] + a_ref[1]) + b_ref[...]
        h = jnp.maximum(z, 0.0)
        hw = dis * jnp.dot(h, w_ref[...], preferred_element_type=jnp.float32)
        o_ref[...] = jnp.concatenate([hw, jnp.zeros_like(hw)], axis=1)

    return pl.pallas_call(
        body,
        grid=(N // _RB,),
        in_specs=[
            pl.BlockSpec((2, _RB, 128), lambda i: (0, i, 0)),
            pl.BlockSpec((2, _RB, 1), lambda i: (0, i, 0)),
            pl.BlockSpec((128,), lambda i: (0,)),
            pl.BlockSpec((128, 64), lambda i: (0, 0)),
        ],
        out_specs=pl.BlockSpec((_RB, 128), lambda i: (i, 0)),
        out_shape=jax.ShapeDtypeStruct((N, 128), jnp.float32),
    )(acc, deg_col, b1, W2)


def _final_out(acc, deg_col, b2):
    def body(a_ref, d_ref, b_ref, o_ref):
        dis = lax.rsqrt(d_ref[0] + d_ref[1])
        z = dis * (a_ref[0, :, :64] + a_ref[1, :, :64]) + b_ref[...]
        m = jnp.max(z, axis=1, keepdims=True)
        s = z - m
        o_ref[...] = s - jnp.log(jnp.sum(jnp.exp(s), axis=1, keepdims=True))

    return pl.pallas_call(
        body,
        grid=(N // _RB,),
        in_specs=[
            pl.BlockSpec((2, _RB, 128), lambda i: (0, i, 0)),
            pl.BlockSpec((2, _RB, 1), lambda i: (0, i, 0)),
            pl.BlockSpec((64,), lambda i: (0,)),
        ],
        out_specs=pl.BlockSpec((_RB, 64), lambda i: (i, 0)),
        out_shape=jax.ShapeDtypeStruct((N, 64), jnp.float32),
    )(acc, deg_col, b2)


def kernel(x, edge_index, edge_weight, W1, b1, W2, b2):
    row = edge_index[0].astype(jnp.int32)
    col = edge_index[1].astype(jnp.int32)
    loop = jnp.arange(N, dtype=jnp.int32)
    pad = EP - (E + N)

    zi = jnp.zeros((pad,), jnp.int32)
    zf = jnp.zeros((pad,), jnp.float32)
    padw = ((0, 0), (0, GPA - GPW), (0, 0))
    row3 = jnp.pad(jnp.concatenate([row, loop, zi]).reshape(NW, GPW, G), padw)
    col3 = jnp.pad(jnp.concatenate([col, loop, zi]).reshape(NW, GPW, G), padw)
    ew3 = jnp.pad(jnp.concatenate([edge_weight, jnp.ones((N,), jnp.float32),
                                   zf]).reshape(NW, GPW, G), padw)
    rowf = row3.reshape(EPA)
    colf = col3.reshape(EPA)
    ewf = ew3.reshape(EPA)

    deg_col = _deg(colf, ewf).reshape(NC, N_PAD, 1)
    h1 = _matmul_scale(x, W1, deg_col)
    acc1 = _spmm128(rowf, col3, ewf, h1)
    h2 = _layer1_out(acc1, deg_col, b1, W2)
    acc2 = _spmm128(rowf, col3, ewf, h2)
    return _final_out(acc2, deg_col, b2)
